# baseline (speedup 1.0000x reference)
import functools

import jax
import jax.numpy as jnp
from jax import lax
from jax.experimental import pallas as pl
from jax.experimental.pallas import tpu as pltpu
from jax.experimental.pallas import tpu_sc as plsc

N = 10000
E = 320000
D = 128
NC = 2
NS = 16
CHUNK = 128
NBUF = 2
GSIZE = 16
CH_PROC = 157
GROUPS = 9
TAIL = CH_PROC - GROUPS * GSIZE
CH = 160
E_PAD_TILE = CH * CHUNK
E_PAD = NS * E_PAD_TILE
ACC_ROWS = 10112
ROWS_PER_TILE = ACC_ROWS // NS


def _sc_segsum_body(with_count, *refs):
    if with_count:
        (table_hbm, src_hbm, dst_hbm, zacc_hbm, zcnt_hbm,
         acc_out, cnt_out,
         src_v, dst_v, rows_v, ones_v, acc_sh, cnt_sh, *sems) = refs
        gsems = sems[:NBUF]
        ssems = sems[NBUF:2 * NBUF]
        csems = sems[2 * NBUF:3 * NBUF]
        isems = sems[3 * NBUF:]
    else:
        (table_hbm, src_hbm, dst_hbm, zacc_hbm,
         acc_out,
         src_v, dst_v, rows_v, acc_sh, *sems) = refs
        gsems = sems[:NBUF]
        ssems = sems[NBUF:2 * NBUF]
        isems = sems[2 * NBUF:]

    c = lax.axis_index("c")
    s = lax.axis_index("s")
    row0 = s * ROWS_PER_TILE

    pltpu.sync_copy(zacc_hbm.at[pl.ds(row0, ROWS_PER_TILE)],
                    acc_sh.at[pl.ds(row0, ROWS_PER_TILE)])
    if with_count:
        @pl.when(s == 0)
        def _():
            pltpu.sync_copy(zcnt_hbm, cnt_sh)
        for i in range(CHUNK // 16):
            ones_v[pl.ds(i * 16, 16)] = jnp.ones((16,), jnp.float32)
    plsc.subcore_barrier()

    def fire_gather(gp, b):
        pltpu.async_copy(table_hbm.at[src_v.at[gp, b]],
                         rows_v.at[b % NBUF], gsems[b % NBUF])

    def wait_gather(gp, b):
        pltpu.make_async_copy(table_hbm.at[src_v.at[gp, b]],
                              rows_v.at[b % NBUF], gsems[b % NBUF]).wait()

    def fire_scatter(gp, b):
        pltpu.async_copy(rows_v.at[b % NBUF], acc_sh.at[dst_v.at[gp, b]],
                         ssems[b % NBUF], add=True)
        if with_count:
            pltpu.async_copy(ones_v, cnt_sh.at[dst_v.at[gp, b]],
                             csems[b % NBUF], add=True)

    def wait_scatter(gp, b):
        pltpu.make_async_copy(rows_v.at[b % NBUF], acc_sh.at[dst_v.at[gp, b]],
                              ssems[b % NBUF]).wait()
        if with_count:
            pltpu.make_async_copy(ones_v, cnt_sh.at[dst_v.at[gp, b]],
                                  csems[b % NBUF]).wait()

    def stage_idx(g, gp):
        pltpu.async_copy(src_hbm.at[c, s].at[pl.ds(g * GSIZE, GSIZE)],
                         src_v.at[gp], isems[0])
        pltpu.async_copy(dst_hbm.at[c, s].at[pl.ds(g * GSIZE, GSIZE)],
                         dst_v.at[gp], isems[1])

    def wait_idx(g, gp):
        pltpu.make_async_copy(src_hbm.at[c, s].at[pl.ds(g * GSIZE, GSIZE)],
                              src_v.at[gp], isems[0]).wait()
        pltpu.make_async_copy(dst_hbm.at[c, s].at[pl.ds(g * GSIZE, GSIZE)],
                              dst_v.at[gp], isems[1]).wait()

    stage_idx(0, 0)
    wait_idx(0, 0)
    fire_gather(0, 0)

    def group_step(g, carry):
        gp = lax.rem(g, 2)
        gn = 1 - gp

        @pl.when(g >= 1)
        def _():
            wait_scatter(gp, 1)

        stage_idx(g + 1, gn)

        for b in range(GSIZE):
            if b + 1 < GSIZE:
                if b >= 1:
                    wait_scatter(gp, b - 1)
                fire_gather(gp, b + 1)
            else:
                wait_idx(g + 1, gn)
                wait_scatter(gp, b - 1)
                fire_gather(gn, 0)
            wait_gather(gp, b)
            fire_scatter(gp, b)
        return carry

    lax.fori_loop(0, GROUPS, group_step, 0)

    gpt = GROUPS % 2
    for b in range(TAIL):
        if b + 1 < TAIL:
            wait_scatter(gpt, b - 1 if b >= 1 else 1)
            fire_gather(gpt, b + 1)
        wait_gather(gpt, b)
        fire_scatter(gpt, b)
    wait_scatter(gpt, TAIL - 2)
    wait_scatter(gpt, TAIL - 1)
    plsc.subcore_barrier()

    pltpu.sync_copy(acc_sh.at[pl.ds(row0, ROWS_PER_TILE)],
                    acc_out.at[c].at[pl.ds(row0, ROWS_PER_TILE)])
    if with_count:
        @pl.when(s == 0)
        def _():
            pltpu.sync_copy(cnt_sh, cnt_out.at[pl.ds(c * ACC_ROWS, ACC_ROWS)])


def _make_sc_segsum(with_count):
    mesh = plsc.VectorSubcoreMesh(core_axis_name="c", subcore_axis_name="s")
    out_type = [jax.ShapeDtypeStruct((NC, ACC_ROWS, D), jnp.float32)]
    scratch = [
        pltpu.VMEM((2, GSIZE, CHUNK), jnp.int32),
        pltpu.VMEM((2, GSIZE, CHUNK), jnp.int32),
        pltpu.VMEM((NBUF, CHUNK, D), jnp.float32),
    ]
    if with_count:
        out_type.append(jax.ShapeDtypeStruct((NC * ACC_ROWS,), jnp.float32))
        scratch.append(pltpu.VMEM((CHUNK,), jnp.float32))
    scratch.append(pltpu.VMEM_SHARED((ACC_ROWS, D), jnp.float32))
    if with_count:
        scratch.append(pltpu.VMEM_SHARED((ACC_ROWS,), jnp.float32))
    n_sems = NBUF * (3 if with_count else 2) + 2
    for _ in range(n_sems):
        scratch.append(pltpu.SemaphoreType.DMA)
    return pl.kernel(
        functools.partial(_sc_segsum_body, with_count),
        out_type=tuple(out_type) if with_count else out_type[0],
        mesh=mesh,
        scratch_types=scratch,
    )


_sc_segsum_l1 = _make_sc_segsum(True)
_sc_segsum_l2 = _make_sc_segsum(False)


_DN = (((1,), (1,)), ((), ()))


def _sage_update(agg, cnt, x, w_l, b, w_r):
    scale = 1.0 / jnp.maximum(cnt, 1.0)
    h = lax.dot_general(agg * scale, w_l[...], _DN,
                        preferred_element_type=jnp.float32)
    return h + b[...] + lax.dot_general(x, w_r[...], _DN,
                                        preferred_element_type=jnp.float32)


def _tc1_body(acc, cnt, x_u, x_m, wu_l, bu, wu_r, wm_l, bm, wm_r, res):
    h_u = _sage_update(acc[1], cnt[1], x_u[...], wu_l, bu, wu_r)
    res[0] = x_u[...] + jnp.maximum(h_u, 0.0)
    h_m = _sage_update(acc[0], cnt[0], x_m[...], wm_l, bm, wm_r)
    res[1] = x_m[...] + jnp.maximum(h_m, 0.0)


def _tc2_body(acc, cnt, res, wu_l, bu, wu_r, wm_l, bm, wm_r, out_u, out_m):
    out_u[...] = _sage_update(acc[1], cnt[1], res[0], wu_l, bu, wu_r)
    out_m[...] = _sage_update(acc[0], cnt[0], res[1], wm_l, bm, wm_r)


_NB = 10
_ROWS = N // _NB
_ACC_SPEC = pl.BlockSpec((NC, _ROWS, D), lambda i: (0, i, 0))
_CNT_SPEC = pl.BlockSpec((NC, _ROWS, 1), lambda i: (0, i, 0))
_PAIR_SPEC = pl.BlockSpec((NC, _ROWS, D), lambda i: (0, i, 0))
_ROW_SPEC = pl.BlockSpec((_ROWS, D), lambda i: (i, 0))
_W_SPEC = pl.BlockSpec((D, D), lambda i: (0, 0))
_B_SPEC = pl.BlockSpec((1, D), lambda i: (0, 0))

_tc_layer1 = pl.pallas_call(
    _tc1_body,
    grid=(_NB,),
    in_specs=[_ACC_SPEC, _CNT_SPEC, _ROW_SPEC, _ROW_SPEC,
              _W_SPEC, _B_SPEC, _W_SPEC, _W_SPEC, _B_SPEC, _W_SPEC],
    out_specs=_PAIR_SPEC,
    out_shape=jax.ShapeDtypeStruct((NC, N, D), jnp.float32),
)

_tc_layer2 = pl.pallas_call(
    _tc2_body,
    grid=(_NB,),
    in_specs=[_ACC_SPEC, _CNT_SPEC, _PAIR_SPEC,
              _W_SPEC, _B_SPEC, _W_SPEC, _W_SPEC, _B_SPEC, _W_SPEC],
    out_specs=[_ROW_SPEC, _ROW_SPEC],
    out_shape=[jax.ShapeDtypeStruct((N, D), jnp.float32),
               jax.ShapeDtypeStruct((N, D), jnp.float32)],
)


def kernel(x_user, x_movie, edge_index_rates, edge_index_rev_rates,
           w1r_l, b1r, w1r_r, w1u_l, b1u, w1u_r,
           w2r_l, b2r, w2r_r, w2u_l, b2u, w2u_r):
    pad_t = E_PAD_TILE - E // NS
    pad_src = jnp.zeros((NS, pad_t), jnp.int32)
    pad_dst = jnp.full((NS, pad_t), N, jnp.int32)

    def tile_layout(idx, pad):
        return jnp.concatenate([idx.reshape(NS, E // NS), pad], axis=1)

    src0 = edge_index_rates[0].astype(jnp.int32)
    dst0 = edge_index_rates[1].astype(jnp.int32)
    src1 = edge_index_rev_rates[0].astype(jnp.int32) + N
    dst1 = edge_index_rev_rates[1].astype(jnp.int32)

    src_g = jnp.stack([tile_layout(src0, pad_src),
                       tile_layout(src1, pad_src)]).reshape(NC, NS, CH, CHUNK)
    dst_g = jnp.stack([tile_layout(dst0, pad_dst),
                       tile_layout(dst1, pad_dst)]).reshape(NC, NS, CH, CHUNK)

    zacc = jnp.zeros((ACC_ROWS, D), jnp.float32)
    zcnt = jnp.zeros((ACC_ROWS,), jnp.float32)

    table1 = jnp.concatenate([x_user, x_movie], axis=0)
    acc1, cnt1 = _sc_segsum_l1(table1, src_g, dst_g, zacc, zcnt)
    cnt3 = cnt1.reshape(NC, ACC_ROWS, 1)

    res = _tc_layer1(acc1, cnt3, x_user, x_movie,
                     w1u_l, b1u[None, :], w1u_r, w1r_l, b1r[None, :], w1r_r)

    table2 = res.reshape(NC * N, D)
    acc2 = _sc_segsum_l2(table2, src_g, dst_g, zacc)

    out_user, out_movie = _tc_layer2(
        acc2, cnt3, res,
        w2u_l, b2u[None, :], w2u_r, w2r_l, b2r[None, :], w2r_r)

    return out_user, out_movie


# speedup vs baseline: 1.0035x; 1.0035x over previous
import functools

import jax
import jax.numpy as jnp
from jax import lax
from jax.experimental import pallas as pl
from jax.experimental.pallas import tpu as pltpu
from jax.experimental.pallas import tpu_sc as plsc

N = 10000
E = 320000
D = 128
NC = 2
NS = 16
CHUNK = 128
NBUF = 2
GSIZE = 8
CH_PROC = 157
GROUPS = 19
TAIL = CH_PROC - GROUPS * GSIZE
CH = 160
E_PAD_TILE = CH * CHUNK
E_PAD = NS * E_PAD_TILE
ACC_ROWS = 10112
ROWS_PER_TILE = ACC_ROWS // NS


def _sc_segsum_body(with_count, *refs):
    if with_count:
        (table_hbm, src_hbm, dst_hbm, zacc_hbm, zcnt_hbm,
         acc_out, cnt_out,
         src_v, dst_v, rows_v, ones_v, acc_sh, cnt_sh, *sems) = refs
        gsems = sems[:NBUF]
        ssems = sems[NBUF:2 * NBUF]
        csems = sems[2 * NBUF:3 * NBUF]
        isems = sems[3 * NBUF:]
    else:
        (table_hbm, src_hbm, dst_hbm, zacc_hbm,
         acc_out,
         src_v, dst_v, rows_v, acc_sh, *sems) = refs
        gsems = sems[:NBUF]
        ssems = sems[NBUF:2 * NBUF]
        isems = sems[2 * NBUF:]

    c = lax.axis_index("c")
    s = lax.axis_index("s")
    row0 = s * ROWS_PER_TILE

    pltpu.sync_copy(zacc_hbm.at[pl.ds(row0, ROWS_PER_TILE)],
                    acc_sh.at[pl.ds(row0, ROWS_PER_TILE)])
    if with_count:
        @pl.when(s == 0)
        def _():
            pltpu.sync_copy(zcnt_hbm, cnt_sh)
        for i in range(CHUNK // 16):
            ones_v[pl.ds(i * 16, 16)] = jnp.ones((16,), jnp.float32)
    plsc.subcore_barrier()

    def fire_gather(gp, b):
        pltpu.async_copy(table_hbm.at[src_v.at[gp, b]],
                         rows_v.at[b % NBUF], gsems[b % NBUF])

    def wait_gather(gp, b):
        pltpu.make_async_copy(table_hbm.at[src_v.at[gp, b]],
                              rows_v.at[b % NBUF], gsems[b % NBUF]).wait()

    def fire_scatter(gp, b):
        pltpu.async_copy(rows_v.at[b % NBUF], acc_sh.at[dst_v.at[gp, b]],
                         ssems[b % NBUF], add=True)
        if with_count:
            pltpu.async_copy(ones_v, cnt_sh.at[dst_v.at[gp, b]],
                             csems[b % NBUF], add=True)

    def wait_scatter(gp, b):
        pltpu.make_async_copy(rows_v.at[b % NBUF], acc_sh.at[dst_v.at[gp, b]],
                              ssems[b % NBUF]).wait()
        if with_count:
            pltpu.make_async_copy(ones_v, cnt_sh.at[dst_v.at[gp, b]],
                                  csems[b % NBUF]).wait()

    def stage_idx(g, gp):
        pltpu.async_copy(src_hbm.at[c, s].at[pl.ds(g * GSIZE, GSIZE)],
                         src_v.at[gp], isems[0])
        pltpu.async_copy(dst_hbm.at[c, s].at[pl.ds(g * GSIZE, GSIZE)],
                         dst_v.at[gp], isems[1])

    def wait_idx(g, gp):
        pltpu.make_async_copy(src_hbm.at[c, s].at[pl.ds(g * GSIZE, GSIZE)],
                              src_v.at[gp], isems[0]).wait()
        pltpu.make_async_copy(dst_hbm.at[c, s].at[pl.ds(g * GSIZE, GSIZE)],
                              dst_v.at[gp], isems[1]).wait()

    stage_idx(0, 0)
    wait_idx(0, 0)
    fire_gather(0, 0)

    def group_step(g, carry):
        gp = lax.rem(g, 2)
        gn = 1 - gp

        @pl.when(g >= 1)
        def _():
            wait_scatter(gp, 1)

        stage_idx(g + 1, gn)

        for b in range(GSIZE):
            if b + 1 < GSIZE:
                if b >= 1:
                    wait_scatter(gp, b - 1)
                fire_gather(gp, b + 1)
            else:
                wait_idx(g + 1, gn)
                wait_scatter(gp, b - 1)
                fire_gather(gn, 0)
            wait_gather(gp, b)
            fire_scatter(gp, b)
        return carry

    lax.fori_loop(0, GROUPS, group_step, 0)

    gpt = GROUPS % 2
    for b in range(TAIL):
        if b + 1 < TAIL:
            wait_scatter(gpt, b - 1 if b >= 1 else 1)
            fire_gather(gpt, b + 1)
        wait_gather(gpt, b)
        fire_scatter(gpt, b)
    wait_scatter(gpt, TAIL - 2)
    wait_scatter(gpt, TAIL - 1)
    plsc.subcore_barrier()

    pltpu.sync_copy(acc_sh.at[pl.ds(row0, ROWS_PER_TILE)],
                    acc_out.at[c].at[pl.ds(row0, ROWS_PER_TILE)])
    if with_count:
        @pl.when(s == 0)
        def _():
            pltpu.sync_copy(cnt_sh, cnt_out.at[pl.ds(c * ACC_ROWS, ACC_ROWS)])


def _make_sc_segsum(with_count):
    mesh = plsc.VectorSubcoreMesh(core_axis_name="c", subcore_axis_name="s")
    out_type = [jax.ShapeDtypeStruct((NC, ACC_ROWS, D), jnp.float32)]
    scratch = [
        pltpu.VMEM((2, GSIZE, CHUNK), jnp.int32),
        pltpu.VMEM((2, GSIZE, CHUNK), jnp.int32),
        pltpu.VMEM((NBUF, CHUNK, D), jnp.float32),
    ]
    if with_count:
        out_type.append(jax.ShapeDtypeStruct((NC * ACC_ROWS,), jnp.float32))
        scratch.append(pltpu.VMEM((CHUNK,), jnp.float32))
    scratch.append(pltpu.VMEM_SHARED((ACC_ROWS, D), jnp.float32))
    if with_count:
        scratch.append(pltpu.VMEM_SHARED((ACC_ROWS,), jnp.float32))
    n_sems = NBUF * (3 if with_count else 2) + 2
    for _ in range(n_sems):
        scratch.append(pltpu.SemaphoreType.DMA)
    return pl.kernel(
        functools.partial(_sc_segsum_body, with_count),
        out_type=tuple(out_type) if with_count else out_type[0],
        mesh=mesh,
        scratch_types=scratch,
    )


_sc_segsum_l1 = _make_sc_segsum(True)
_sc_segsum_l2 = _make_sc_segsum(False)


_DN = (((1,), (1,)), ((), ()))


def _sage_update(agg, cnt, x, w_l, b, w_r):
    scale = 1.0 / jnp.maximum(cnt, 1.0)
    h = lax.dot_general(agg * scale, w_l[...], _DN,
                        preferred_element_type=jnp.float32)
    return h + b[...] + lax.dot_general(x, w_r[...], _DN,
                                        preferred_element_type=jnp.float32)


def _tc1_body(acc, cnt, x_u, x_m, wu_l, bu, wu_r, wm_l, bm, wm_r, res):
    h_u = _sage_update(acc[1], cnt[1], x_u[...], wu_l, bu, wu_r)
    res[0] = x_u[...] + jnp.maximum(h_u, 0.0)
    h_m = _sage_update(acc[0], cnt[0], x_m[...], wm_l, bm, wm_r)
    res[1] = x_m[...] + jnp.maximum(h_m, 0.0)


def _tc2_body(acc, cnt, res, wu_l, bu, wu_r, wm_l, bm, wm_r, out_u, out_m):
    out_u[...] = _sage_update(acc[1], cnt[1], res[0], wu_l, bu, wu_r)
    out_m[...] = _sage_update(acc[0], cnt[0], res[1], wm_l, bm, wm_r)


_NB = 10
_ROWS = N // _NB
_ACC_SPEC = pl.BlockSpec((NC, _ROWS, D), lambda i: (0, i, 0))
_CNT_SPEC = pl.BlockSpec((NC, _ROWS, 1), lambda i: (0, i, 0))
_PAIR_SPEC = pl.BlockSpec((NC, _ROWS, D), lambda i: (0, i, 0))
_ROW_SPEC = pl.BlockSpec((_ROWS, D), lambda i: (i, 0))
_W_SPEC = pl.BlockSpec((D, D), lambda i: (0, 0))
_B_SPEC = pl.BlockSpec((1, D), lambda i: (0, 0))

_tc_layer1 = pl.pallas_call(
    _tc1_body,
    grid=(_NB,),
    in_specs=[_ACC_SPEC, _CNT_SPEC, _ROW_SPEC, _ROW_SPEC,
              _W_SPEC, _B_SPEC, _W_SPEC, _W_SPEC, _B_SPEC, _W_SPEC],
    out_specs=_PAIR_SPEC,
    out_shape=jax.ShapeDtypeStruct((NC, N, D), jnp.float32),
)

_tc_layer2 = pl.pallas_call(
    _tc2_body,
    grid=(_NB,),
    in_specs=[_ACC_SPEC, _CNT_SPEC, _PAIR_SPEC,
              _W_SPEC, _B_SPEC, _W_SPEC, _W_SPEC, _B_SPEC, _W_SPEC],
    out_specs=[_ROW_SPEC, _ROW_SPEC],
    out_shape=[jax.ShapeDtypeStruct((N, D), jnp.float32),
               jax.ShapeDtypeStruct((N, D), jnp.float32)],
)


def kernel(x_user, x_movie, edge_index_rates, edge_index_rev_rates,
           w1r_l, b1r, w1r_r, w1u_l, b1u, w1u_r,
           w2r_l, b2r, w2r_r, w2u_l, b2u, w2u_r):
    pad_t = E_PAD_TILE - E // NS
    pad_src = jnp.zeros((NS, pad_t), jnp.int32)
    pad_dst = jnp.full((NS, pad_t), N, jnp.int32)

    def tile_layout(idx, pad):
        return jnp.concatenate([idx.reshape(NS, E // NS), pad], axis=1)

    src0 = edge_index_rates[0].astype(jnp.int32)
    dst0 = edge_index_rates[1].astype(jnp.int32)
    src1 = edge_index_rev_rates[0].astype(jnp.int32) + N
    dst1 = edge_index_rev_rates[1].astype(jnp.int32)

    src_g = jnp.stack([tile_layout(src0, pad_src),
                       tile_layout(src1, pad_src)]).reshape(NC, NS, CH, CHUNK)
    dst_g = jnp.stack([tile_layout(dst0, pad_dst),
                       tile_layout(dst1, pad_dst)]).reshape(NC, NS, CH, CHUNK)

    zacc = jnp.zeros((ACC_ROWS, D), jnp.float32)
    zcnt = jnp.zeros((ACC_ROWS,), jnp.float32)

    table1 = jnp.concatenate([x_user, x_movie], axis=0)
    acc1, cnt1 = _sc_segsum_l1(table1, src_g, dst_g, zacc, zcnt)
    cnt3 = cnt1.reshape(NC, ACC_ROWS, 1)

    res = _tc_layer1(acc1, cnt3, x_user, x_movie,
                     w1u_l, b1u[None, :], w1u_r, w1r_l, b1r[None, :], w1r_r)

    table2 = res.reshape(NC * N, D)
    acc2 = _sc_segsum_l2(table2, src_g, dst_g, zacc)

    out_user, out_movie = _tc_layer2(
        acc2, cnt3, res,
        w2u_l, b2u[None, :], w2u_r, w2r_l, b2r[None, :], w2r_r)

    return out_user, out_movie
